# baseline (speedup 1.0000x reference)
import jax
import jax.numpy as jnp
from jax.experimental import pallas as pl

_B, _C, _H, _W = 16, 16, 320, 320
_R, _L = 800, 128
_CB = 16
_HALF = 128


def _copy_body(a_ref, b_ref, oa_ref, ob_ref):
    oa_ref[...] = a_ref[...]
    ob_ref[...] = b_ref[...]


def kernel(kspace, weights):
    ks = kspace.reshape(_B * _C, _R, _L)
    outa, outb = pl.pallas_call(
        _copy_body,
        grid=(_HALF // _CB,),
        in_specs=[
            pl.BlockSpec((_CB, _R, _L), lambda i: (i, 0, 0)),
            pl.BlockSpec((_CB, _R, _L), lambda i: (i + _HALF // _CB, 0, 0)),
        ],
        out_specs=[
            pl.BlockSpec((_CB, _R, _L), lambda i: (i, 0, 0)),
            pl.BlockSpec((_CB, _R, _L), lambda i: (i, 0, 0)),
        ],
        out_shape=[
            jax.ShapeDtypeStruct((_HALF, _R, _L), jnp.float32),
            jax.ShapeDtypeStruct((_HALF, _R, _L), jnp.float32),
        ],
    )(ks, ks)
    out = jnp.concatenate([outa, outb], axis=0)
    mout = jnp.zeros((_B, _H, _W), jnp.float32)
    return out.reshape(_B, _C, _H, _W), mout


# speedup vs baseline: 1.0103x; 1.0103x over previous
import functools

import jax
import jax.numpy as jnp
from jax import lax
from jax.experimental import pallas as pl
from jax.experimental.pallas import tpu as pltpu
from jax.experimental.pallas import tpu_sc as plsc

_B, _C, _H, _W = 16, 16, 320, 320
_HW = _H * _W
_R, _L = 800, 128
_K = 25600
_SP = 0.25
_ONE_BITS = 0x3F800001

_NROW = _B * _C
_NW = 32
_CPT = _HW // _NW
_RCH = 8
_NCH = _NROW // _RCH


def _mask_body(w_ref, mask_ref, mout_ref):
    x = w_ref[...]
    prob = jax.nn.sigmoid(x)
    xbar = jnp.mean(prob)
    r = _SP / xbar
    beta = (1.0 - _SP) / (1.0 - xbar)
    le = (r <= 1.0).astype(jnp.float32)
    resc = le * (prob * r) + (1.0 - le) * (1.0 - (1.0 - prob) * beta)
    bits = lax.bitcast_convert_type(resc, jnp.int32)

    def body(_, lohi):
        lo, hi = lohi
        mid = (lo + hi) // 2
        cnt = jnp.sum((bits >= mid).astype(jnp.int32))
        ok = cnt >= _K
        return jnp.where(ok, mid, lo), jnp.where(ok, hi, mid)

    lo, _hi = lax.fori_loop(0, 31, body, (jnp.int32(0), jnp.int32(_ONE_BITS)))
    m = (bits >= lo).astype(jnp.float32)
    mask_ref[...] = m
    mout_ref[...] = jnp.broadcast_to(m[None], (_B, _R, _L))


_sc_mesh = plsc.VectorSubcoreMesh(core_axis_name="c", subcore_axis_name="s")


@functools.partial(
    pl.kernel,
    mesh=_sc_mesh,
    out_type=jax.ShapeDtypeStruct((_NROW, _HW), jnp.float32),
    scratch_types=[
        pltpu.VMEM((_RCH, _CPT), jnp.float32),
        pltpu.VMEM((_RCH, _CPT), jnp.float32),
        pltpu.VMEM((_CPT,), jnp.float32),
    ],
)
def _sc_mul(ks_hbm, mask_hbm, out_hbm, ibuf, obuf, mbuf):
    wid = lax.axis_index("s") * 2 + lax.axis_index("c")
    c0 = wid * _CPT
    pltpu.sync_copy(mask_hbm.at[pl.ds(c0, _CPT)], mbuf)

    def chunk(c, carry):
        r0 = c * _RCH
        pltpu.sync_copy(ks_hbm.at[pl.ds(r0, _RCH), pl.ds(c0, _CPT)], ibuf)

        def inner(l, carry2):
            s = pl.multiple_of(l * 16, 16)
            m = mbuf[pl.ds(s, 16)]
            for r in range(_RCH):
                obuf[r, pl.ds(s, 16)] = ibuf[r, pl.ds(s, 16)] * m
            return carry2

        lax.fori_loop(0, _CPT // 16, inner, 0)
        pltpu.sync_copy(obuf, out_hbm.at[pl.ds(r0, _RCH), pl.ds(c0, _CPT)])
        return carry

    lax.fori_loop(0, _NCH, chunk, 0)


def kernel(kspace, weights):
    ks = kspace.reshape(_NROW, _HW)
    w = weights.reshape(_R, _L)
    mask2d, mout = pl.pallas_call(
        _mask_body,
        in_specs=[pl.BlockSpec((_R, _L), lambda: (0, 0))],
        out_specs=[
            pl.BlockSpec((_R, _L), lambda: (0, 0)),
            pl.BlockSpec((_B, _R, _L), lambda: (0, 0, 0)),
        ],
        out_shape=[
            jax.ShapeDtypeStruct((_R, _L), jnp.float32),
            jax.ShapeDtypeStruct((_B, _R, _L), jnp.float32),
        ],
    )(w)
    out = _sc_mul(ks, mask2d.reshape(_HW))
    return out.reshape(_B, _C, _H, _W), mout.reshape(_B, _H, _W)


# speedup vs baseline: 1.3064x; 1.2930x over previous
import functools

import jax
import jax.numpy as jnp
from jax import lax
from jax.experimental import pallas as pl
from jax.experimental.pallas import tpu as pltpu
from jax.experimental.pallas import tpu_sc as plsc

_B, _C, _H, _W = 16, 16, 320, 320
_HW = _H * _W
_R, _L = 800, 128
_K = 25600
_SP = 0.25
_ONE_BITS = 0x3F800001

_NROW = _B * _C
_NW = 32
_CPT = _HW // _NW
_RCH = 8
_NCH = _NROW // _RCH


def _mask_body(w_ref, mask_ref, mout_ref):
    x = w_ref[...]
    prob = jax.nn.sigmoid(x)
    xbar = jnp.mean(prob)
    r = _SP / xbar
    beta = (1.0 - _SP) / (1.0 - xbar)
    le = (r <= 1.0).astype(jnp.float32)
    resc = le * (prob * r) + (1.0 - le) * (1.0 - (1.0 - prob) * beta)
    bits = lax.bitcast_convert_type(resc, jnp.int32)

    def body(_, lohi):
        lo, hi = lohi
        mid = (lo + hi) // 2
        cnt = jnp.sum((bits >= mid).astype(jnp.int32))
        ok = cnt >= _K
        return jnp.where(ok, mid, lo), jnp.where(ok, hi, mid)

    lo, _hi = lax.fori_loop(0, 31, body, (jnp.int32(0), jnp.int32(_ONE_BITS)))
    m = (bits >= lo).astype(jnp.float32)
    mask_ref[...] = m
    mout_ref[...] = jnp.broadcast_to(m[None], (_B, _R, _L))


_sc_mesh = plsc.VectorSubcoreMesh(core_axis_name="c", subcore_axis_name="s")


@functools.partial(
    pl.kernel,
    mesh=_sc_mesh,
    out_type=jax.ShapeDtypeStruct((_NROW, _HW), jnp.float32),
    scratch_types=[
        pltpu.VMEM((_RCH, _CPT), jnp.float32),
        pltpu.VMEM((_RCH, _CPT), jnp.float32),
        pltpu.VMEM((_RCH, _CPT), jnp.float32),
        pltpu.VMEM((_RCH, _CPT), jnp.float32),
        pltpu.VMEM((_CPT,), jnp.float32),
        pltpu.SemaphoreType.DMA,
        pltpu.SemaphoreType.DMA,
        pltpu.SemaphoreType.DMA,
        pltpu.SemaphoreType.DMA,
    ],
)
def _sc_mul(ks_hbm, mask_hbm, out_hbm, ib0, ib1, ob0, ob1, mbuf,
            si0, si1, so0, so1):
    wid = lax.axis_index("s") * 2 + lax.axis_index("c")
    c0 = wid * _CPT
    pltpu.sync_copy(mask_hbm.at[pl.ds(c0, _CPT)], mbuf)

    def src(c):
        return ks_hbm.at[pl.ds(c * _RCH, _RCH), pl.ds(c0, _CPT)]

    def dst(c):
        return out_hbm.at[pl.ds(c * _RCH, _RCH), pl.ds(c0, _CPT)]

    def compute(ib, ob):
        def inner(l, carry2):
            s = pl.multiple_of(l * 16, 16)
            m = mbuf[pl.ds(s, 16)]
            for r in range(_RCH):
                ob[r, pl.ds(s, 16)] = ib[r, pl.ds(s, 16)] * m
            return carry2

        lax.fori_loop(0, _CPT // 16, inner, 0)

    pltpu.async_copy(src(0), ib0, si0)
    pltpu.async_copy(src(1), ib1, si1)

    def pair(i, carry):
        c = i * 2

        @pl.when(i >= 1)
        def _():
            pltpu.make_async_copy(src(0), ob0, so0).wait()

        pltpu.make_async_copy(src(0), ib0, si0).wait()
        compute(ib0, ob0)
        pltpu.async_copy(ob0, dst(c), so0)

        @pl.when(c + 2 < _NCH)
        def _():
            pltpu.async_copy(src(c + 2), ib0, si0)

        @pl.when(i >= 1)
        def _():
            pltpu.make_async_copy(src(0), ob1, so1).wait()

        pltpu.make_async_copy(src(0), ib1, si1).wait()
        compute(ib1, ob1)
        pltpu.async_copy(ob1, dst(c + 1), so1)

        @pl.when(c + 3 < _NCH)
        def _():
            pltpu.async_copy(src(c + 3), ib1, si1)

        return carry

    lax.fori_loop(0, _NCH // 2, pair, 0)
    pltpu.make_async_copy(src(0), ob0, so0).wait()
    pltpu.make_async_copy(src(0), ob1, so1).wait()


def kernel(kspace, weights):
    ks = kspace.reshape(_NROW, _HW)
    w = weights.reshape(_R, _L)
    mask2d, mout = pl.pallas_call(
        _mask_body,
        in_specs=[pl.BlockSpec((_R, _L), lambda: (0, 0))],
        out_specs=[
            pl.BlockSpec((_R, _L), lambda: (0, 0)),
            pl.BlockSpec((_B, _R, _L), lambda: (0, 0, 0)),
        ],
        out_shape=[
            jax.ShapeDtypeStruct((_R, _L), jnp.float32),
            jax.ShapeDtypeStruct((_B, _R, _L), jnp.float32),
        ],
    )(w)
    out = _sc_mul(ks, mask2d.reshape(_HW))
    return out.reshape(_B, _C, _H, _W), mout.reshape(_B, _H, _W)
